# baseline (speedup 1.0000x reference)
import functools

import jax
import jax.numpy as jnp
from jax import lax
from jax.experimental import pallas as pl
from jax.experimental.pallas import tpu as pltpu
from jax.experimental.pallas import tpu_sc as plsc

EMBEDDING_DIM = 1024
N_INDICES = 4 * 8192
N_SC = (N_INDICES // 4) * 3
N_TC = N_INDICES - N_SC

_info = plsc.get_sparse_core_info()
NC, NS = _info.num_cores, _info.num_subcores
NW = NC * NS
PER_W = N_SC // NW
CHUNK = 16
N_CHUNKS = PER_W // CHUNK
NBUF = 4
LAG = 2


def _sc_gather(pe, pos_flat):
    mesh = plsc.VectorSubcoreMesh(core_axis_name="c", subcore_axis_name="s")

    @functools.partial(
        pl.kernel,
        out_type=jax.ShapeDtypeStruct((N_INDICES, EMBEDDING_DIM), jnp.float32),
        mesh=mesh,
        scratch_types=[
            pltpu.VMEM((PER_W,), jnp.int32),
            pltpu.VMEM((NBUF, CHUNK, EMBEDDING_DIM), jnp.float32),
        ] + [pltpu.SemaphoreType.DMA] * (2 * NBUF),
    )
    def k(table_hbm, idx_hbm, out_hbm, idx_v, rows_v, *sems):
        wid = lax.axis_index("s") * NC + lax.axis_index("c")
        base = wid * PER_W
        gsem = sems[:NBUF]
        wsem = sems[NBUF:]

        pltpu.sync_copy(idx_hbm.at[pl.ds(base, PER_W)], idx_v)

        def start_gather(c, b):
            pltpu.async_copy(
                table_hbm.at[idx_v.at[pl.ds(c * CHUNK, CHUNK)]],
                rows_v.at[b], gsem[b])

        def wait_gather(b):
            pltpu.make_async_copy(table_hbm.at[idx_v.at[pl.ds(0, CHUNK)]],
                                  rows_v.at[b], gsem[b]).wait()

        def start_write(c, b):
            pltpu.async_copy(rows_v.at[b],
                             out_hbm.at[pl.ds(base + c * CHUNK, CHUNK)], wsem[b])

        def wait_write(b):
            pltpu.make_async_copy(rows_v.at[b],
                                  out_hbm.at[pl.ds(0, CHUNK)], wsem[b]).wait()

        for j in range(NBUF):
            start_gather(j, j)
            if j >= LAG:
                wait_gather(j - LAG)
                start_write(j - LAG, j - LAG)

        def body(i, carry):
            for j in range(NBUF):
                s = NBUF * i + j
                wait_write(j)
                start_gather(s, j)
                wait_gather((j - LAG) % NBUF)
                start_write(s - LAG, (j - LAG) % NBUF)
            return carry

        lax.fori_loop(1, N_CHUNKS // NBUF, body, 0)

        for c in range(N_CHUNKS - LAG, N_CHUNKS):
            wait_gather(c % NBUF)
            start_write(c, c % NBUF)
        for c in range(N_CHUNKS - NBUF, N_CHUNKS):
            wait_write(c % NBUF)

    return k(pe, pos_flat)


def _tc_gather(pe, idx):
    def body(idx_ref, pe_ref, out_ref):
        out_ref[...] = pe_ref[...]

    grid_spec = pltpu.PrefetchScalarGridSpec(
        num_scalar_prefetch=1,
        grid=(N_TC,),
        in_specs=[pl.BlockSpec((1, 8, 128), lambda i, idx: (idx[i], 0, 0))],
        out_specs=pl.BlockSpec((1, 8, 128), lambda i, idx: (i, 0, 0)),
    )
    out = pl.pallas_call(
        body,
        grid_spec=grid_spec,
        out_shape=jax.ShapeDtypeStruct((N_TC, 8, 128), jnp.float32),
    )(idx, pe.reshape(-1, 8, 128))
    return out.reshape(N_TC, EMBEDDING_DIM)


def kernel(pe, pos):
    pos_flat = pos.reshape(-1).astype(jnp.int32)
    sc_out = _sc_gather(pe, pos_flat)
    tc_part = _tc_gather(pe, pos_flat[N_SC:])
    out = lax.dynamic_update_slice(sc_out, tc_part, (N_SC, 0))
    return out.reshape((*pos.shape, EMBEDDING_DIM))


# speedup vs baseline: 32.7230x; 32.7230x over previous
import functools

import jax
import jax.numpy as jnp
from jax import lax
from jax.experimental import pallas as pl
from jax.experimental.pallas import tpu as pltpu
from jax.experimental.pallas import tpu_sc as plsc

EMBEDDING_DIM = 1024
N_INDICES = 4 * 8192

_info = plsc.get_sparse_core_info()
NC, NS = _info.num_cores, _info.num_subcores
NW = NC * NS
PER_W = N_INDICES // NW
CHUNK = 16
N_CHUNKS = PER_W // CHUNK
NBUF = 4
LAG = 2


def _sc_gather(pe, pos_flat):
    mesh = plsc.VectorSubcoreMesh(core_axis_name="c", subcore_axis_name="s")

    @functools.partial(
        pl.kernel,
        out_type=jax.ShapeDtypeStruct((N_INDICES, EMBEDDING_DIM), jnp.float32),
        mesh=mesh,
        scratch_types=[
            pltpu.VMEM((PER_W,), jnp.int32),
            pltpu.VMEM((NBUF, CHUNK, EMBEDDING_DIM), jnp.float32),
        ] + [pltpu.SemaphoreType.DMA] * (2 * NBUF),
    )
    def k(table_hbm, idx_hbm, out_hbm, idx_v, rows_v, *sems):
        wid = lax.axis_index("s") * NC + lax.axis_index("c")
        base = wid * PER_W
        gsem = sems[:NBUF]
        wsem = sems[NBUF:]

        pltpu.sync_copy(idx_hbm.at[pl.ds(base, PER_W)], idx_v)

        def start_gather(c, b):
            pltpu.async_copy(
                table_hbm.at[idx_v.at[pl.ds(c * CHUNK, CHUNK)]],
                rows_v.at[b], gsem[b])

        def wait_gather(b):
            pltpu.make_async_copy(table_hbm.at[idx_v.at[pl.ds(0, CHUNK)]],
                                  rows_v.at[b], gsem[b]).wait()

        def start_write(c, b):
            pltpu.async_copy(rows_v.at[b],
                             out_hbm.at[pl.ds(base + c * CHUNK, CHUNK)], wsem[b])

        def wait_write(b):
            pltpu.make_async_copy(rows_v.at[b],
                                  out_hbm.at[pl.ds(0, CHUNK)], wsem[b]).wait()

        for j in range(NBUF):
            start_gather(j, j)
            if j >= LAG:
                wait_gather(j - LAG)
                start_write(j - LAG, j - LAG)

        def body(i, carry):
            for j in range(NBUF):
                s = NBUF * i + j
                wait_write(j)
                start_gather(s, j)
                wait_gather((j - LAG) % NBUF)
                start_write(s - LAG, (j - LAG) % NBUF)
            return carry

        lax.fori_loop(1, N_CHUNKS // NBUF, body, 0)

        for c in range(N_CHUNKS - LAG, N_CHUNKS):
            wait_gather(c % NBUF)
            start_write(c, c % NBUF)
        for c in range(N_CHUNKS - NBUF, N_CHUNKS):
            wait_write(c % NBUF)

    return k(pe, pos_flat)


def kernel(pe, pos):
    pos_flat = pos.reshape(-1).astype(jnp.int32)
    out = _sc_gather(pe, pos_flat)
    return out.reshape((*pos.shape, EMBEDDING_DIM))
